# Initial kernel scaffold; baseline (speedup 1.0000x reference)
import jax
import jax.numpy as jnp
from jax.experimental import pallas as pl


def kernel(node_states, segment_ids, context_state, W, b):
    raise NotImplementedError("write your pallas kernel here")


# speedup vs baseline: 4.3678x; 4.3678x over previous
import dataclasses

import jax
import jax.numpy as jnp
from jax import lax
from jax.experimental import pallas as pl
from jax.experimental.pallas import tpu as pltpu
from jax.experimental.pallas import tpu_sc as plsc

N_NODES = 100000
N_SEG = 256
D = 128
CH = 80
NCHUNKS = N_NODES // CH
NC = 2
NS = 16
NW = NC * NS
BASE_CHUNKS = NCHUNKS // NW
EXTRA = NCHUNKS - BASE_CHUNKS * NW
CNT_W = 16


def _sc_body(nodes_hbm, ids_hbm, zeros_hbm, zcnt_hbm, sums_hbm,
             counts_hbm, rows_v, ids_v, cnt_v, acc_sh):
  c = lax.axis_index("c")
  s = lax.axis_index("s")
  wid = c * NS + s

  pltpu.sync_copy(zcnt_hbm, cnt_v)

  @pl.when(s == 0)
  def _():
    pltpu.sync_copy(zeros_hbm, acc_sh)

  plsc.subcore_barrier()

  lane = lax.iota(jnp.int32, 16)
  one16 = jnp.full((16,), 1.0, jnp.float32)

  cnt = jnp.where(wid < EXTRA, BASE_CHUNKS + 1, BASE_CHUNKS)
  start = wid * BASE_CHUNKS + jnp.minimum(wid, EXTRA)

  @pl.loop(0, BASE_CHUNKS + 1)
  def _(t):
    @pl.when(t < cnt)
    def _():
      chunk = start + t
      pltpu.sync_copy(nodes_hbm.at[pl.ds(chunk * CH, CH)], rows_v)
      pltpu.sync_copy(ids_hbm.at[pl.ds(chunk, 1)], ids_v)
      pltpu.sync_copy(rows_v, acc_sh.at[ids_v.at[0]], add=True)
      for k in range(CH // 16):
        idv = ids_v[0, pl.ds(k * 16, 16)]
        plsc.addupdate_scatter(cnt_v, [idv, lane], one16)

  plsc.subcore_barrier()

  @pl.when(s == 0)
  def _():
    pltpu.sync_copy(acc_sh, sums_hbm.at[c])
  pltpu.sync_copy(cnt_v, counts_hbm.at[wid])


@jax.jit
def _sc_segment_sums(nodes, ids2d, zeros, zcnt):
  mesh = plsc.VectorSubcoreMesh(core_axis_name="c", subcore_axis_name="s")
  cp = pltpu.CompilerParams()
  if "needs_layout_passes" in pltpu.CompilerParams.__dataclass_fields__:
    cp = dataclasses.replace(cp, needs_layout_passes=False)
  kern = pl.kernel(
      _sc_body,
      out_type=(
          jax.ShapeDtypeStruct((NC, N_SEG, D), jnp.float32),
          jax.ShapeDtypeStruct((NW, N_SEG, CNT_W), jnp.float32),
      ),
      mesh=mesh,
      scratch_types=[
          pltpu.VMEM((CH, D), jnp.float32),
          pltpu.VMEM((1, CH), jnp.int32),
          pltpu.VMEM((N_SEG, CNT_W), jnp.float32),
          pltpu.VMEM_SHARED((N_SEG, D), jnp.float32),
      ],
      compiler_params=cp,
  )
  return kern(nodes, ids2d, zeros, zcnt)


def _tc_body(sums_ref, counts_ref, ctx_ref, w_ref, b_ref, out_ref):
  sums = sums_ref[0] + sums_ref[1]
  cnt = counts_ref[...].sum(axis=0).sum(axis=-1)[:, None]
  pooled = sums / jnp.maximum(cnt, 1.0)
  w_ctx = w_ref[0:D, :]
  w_pool = w_ref[D:2 * D, :]
  out = (
      lax.dot_general(ctx_ref[...], w_ctx, (((1,), (0,)), ((), ())),
                      preferred_element_type=jnp.float32)
      + lax.dot_general(pooled, w_pool, (((1,), (0,)), ((), ())),
                        preferred_element_type=jnp.float32)
      + b_ref[...]
  )
  out_ref[...] = jnp.maximum(out, 0.0)


@jax.jit
def _tc_finish(sums, counts, context_state, w, b2d):
  return pl.pallas_call(
      _tc_body,
      out_shape=jax.ShapeDtypeStruct((N_SEG, D), jnp.float32),
  )(sums, counts, context_state, w, b2d)


def kernel(node_states, segment_ids, context_state, W, b):
  ids2d = segment_ids.astype(jnp.int32).reshape(NCHUNKS, CH)
  zeros = jnp.zeros((N_SEG, D), jnp.float32)
  zcnt = jnp.zeros((N_SEG, CNT_W), jnp.float32)
  sums, counts = _sc_segment_sums(node_states, ids2d, zeros, zcnt)
  return _tc_finish(sums, counts, context_state, W, b.reshape(1, D))


# speedup vs baseline: 7.6352x; 1.7481x over previous
import dataclasses

import jax
import jax.numpy as jnp
from jax import lax
from jax.experimental import pallas as pl
from jax.experimental.pallas import tpu as pltpu
from jax.experimental.pallas import tpu_sc as plsc

N_NODES = 100000
N_SEG = 256
D = 128
CH = 80
NCHUNKS = N_NODES // CH
NC = 2
NS = 16
NW = NC * NS
CNT_W = 16
MULTI = 4
BIG = MULTI * CH
NBIG = NCHUNKS // MULTI
TAIL = NCHUNKS - NBIG * MULTI
T_STEPS = -(-NBIG // NW)


def _sc_body(nodes_hbm, ids_hbm, zeros_hbm, zcnt_hbm, sums_hbm,
             counts_hbm, rows_v, ids_v, cnt_v, acc_sh,
             sem_r0, sem_r1, sem_i0, sem_i1):
  c = lax.axis_index("c")
  s = lax.axis_index("s")
  wid = c * NS + s
  sem_r = (sem_r0, sem_r1)
  sem_i = (sem_i0, sem_i1)

  pltpu.sync_copy(zcnt_hbm, cnt_v)

  @pl.when(s == 0)
  def _():
    pltpu.sync_copy(zeros_hbm, acc_sh)

  plsc.subcore_barrier()

  lane = lax.iota(jnp.int32, 16)
  one16 = jnp.full((16,), 1.0, jnp.float32)

  def load_descs(t, slot):
    b = wid + NW * t
    return (
        pltpu.make_async_copy(nodes_hbm.at[pl.ds(b * BIG, BIG)],
                              rows_v.at[slot], sem_r[slot]),
        pltpu.make_async_copy(ids_hbm.at[pl.ds(b, 1)],
                              ids_v.at[slot], sem_i[slot]),
    )

  for t in range(T_STEPS):
    slot = t % 2
    if t == 0:
      @pl.when(wid + NW * t < NBIG)
      def _(t=t, slot=slot):
        for d in load_descs(t, slot):
          d.start()
    if t + 1 < T_STEPS:
      @pl.when(wid + NW * (t + 1) < NBIG)
      def _(t=t, slot=slot):
        for d in load_descs(t + 1, 1 - slot):
          d.start()

    @pl.when(wid + NW * t < NBIG)
    def _(t=t, slot=slot):
      for d in load_descs(t, slot):
        d.wait()
      descs = []
      for j in range(MULTI):
        descs.append(pltpu.async_copy(
            rows_v.at[slot, pl.ds(j * CH, CH)],
            acc_sh.at[ids_v.at[slot, 0, j]], sem_r[slot], add=True))
      for j in range(MULTI):
        for k in range(CH // 16):
          idv = ids_v[slot, 0, j, pl.ds(k * 16, 16)]
          plsc.addupdate_scatter(cnt_v, [idv, lane], one16)
      for d in descs:
        d.wait()

  @pl.when(wid < TAIL)
  def _():
    chunk = NBIG * MULTI + wid
    pltpu.sync_copy(nodes_hbm.at[pl.ds(chunk * CH, CH)],
                    rows_v.at[0, pl.ds(0, CH)])
    pltpu.sync_copy(ids_hbm.at[pl.ds(NBIG, 1)], ids_v.at[0])
    pltpu.sync_copy(rows_v.at[0, pl.ds(0, CH)],
                    acc_sh.at[ids_v.at[0, 0, wid]], add=True)
    for k in range(CH // 16):
      idv = ids_v[0, 0, wid, pl.ds(k * 16, 16)]
      plsc.addupdate_scatter(cnt_v, [idv, lane], one16)

  plsc.subcore_barrier()

  @pl.when(s == 0)
  def _():
    pltpu.sync_copy(acc_sh, sums_hbm.at[c])
  pltpu.sync_copy(cnt_v, counts_hbm.at[wid])


@jax.jit
def _sc_segment_sums(nodes, ids2d, zeros, zcnt):
  mesh = plsc.VectorSubcoreMesh(core_axis_name="c", subcore_axis_name="s")
  cp = pltpu.CompilerParams()
  if "needs_layout_passes" in pltpu.CompilerParams.__dataclass_fields__:
    cp = dataclasses.replace(cp, needs_layout_passes=False)
  kern = pl.kernel(
      _sc_body,
      out_type=(
          jax.ShapeDtypeStruct((NC, N_SEG, D), jnp.float32),
          jax.ShapeDtypeStruct((NW, N_SEG, CNT_W), jnp.float32),
      ),
      mesh=mesh,
      scratch_types=[
          pltpu.VMEM((2, BIG, D), jnp.float32),
          pltpu.VMEM((2, 1, MULTI, CH), jnp.int32),
          pltpu.VMEM((N_SEG, CNT_W), jnp.float32),
          pltpu.VMEM_SHARED((N_SEG, D), jnp.float32),
          pltpu.SemaphoreType.DMA,
          pltpu.SemaphoreType.DMA,
          pltpu.SemaphoreType.DMA,
          pltpu.SemaphoreType.DMA,
      ],
      compiler_params=cp,
  )
  return kern(nodes, ids2d, zeros, zcnt)


def _tc_body(sums_ref, counts_ref, ctx_ref, w_ref, b_ref, out_ref):
  sums = sums_ref[0] + sums_ref[1]
  cnt = counts_ref[...].sum(axis=0).sum(axis=-1)[:, None]
  pooled = sums / jnp.maximum(cnt, 1.0)
  w_ctx = w_ref[0:D, :]
  w_pool = w_ref[D:2 * D, :]
  out = (
      lax.dot_general(ctx_ref[...], w_ctx, (((1,), (0,)), ((), ())),
                      preferred_element_type=jnp.float32)
      + lax.dot_general(pooled, w_pool, (((1,), (0,)), ((), ())),
                        preferred_element_type=jnp.float32)
      + b_ref[...]
  )
  out_ref[...] = jnp.maximum(out, 0.0)


@jax.jit
def _tc_finish(sums, counts, context_state, w, b2d):
  return pl.pallas_call(
      _tc_body,
      out_shape=jax.ShapeDtypeStruct((N_SEG, D), jnp.float32),
  )(sums, counts, context_state, w, b2d)


def kernel(node_states, segment_ids, context_state, W, b):
  ids_flat = segment_ids.astype(jnp.int32)
  pad = (NBIG + 1) * MULTI * CH - N_NODES
  ids2d = jnp.concatenate(
      [ids_flat, jnp.zeros((pad,), jnp.int32)]).reshape(NBIG + 1, MULTI, CH)
  zeros = jnp.zeros((N_SEG, D), jnp.float32)
  zcnt = jnp.zeros((N_SEG, CNT_W), jnp.float32)
  sums, counts = _sc_segment_sums(node_states, ids2d, zeros, zcnt)
  return _tc_finish(sums, counts, context_state, W, b.reshape(1, D))
